# Initial kernel scaffold; baseline (speedup 1.0000x reference)
import jax
import jax.numpy as jnp
from jax.experimental import pallas as pl


def kernel(features, edge_index, W1, al1, ar1, b1, W2, al2, ar2, b2):
    raise NotImplementedError("write your pallas kernel here")


# speedup vs baseline: 12.5978x; 12.5978x over previous
import functools

import jax
import jax.numpy as jnp
from jax import lax
from jax.experimental import pallas as pl
from jax.experimental.pallas import tpu as pltpu
from jax.experimental.pallas import tpu_sc as plsc

N = 10000
NPAD = 10240
E = 320000
ROWS = 2560
EPAD = ROWS * 128
IN = 128
H = 8
D = 64
HD = H * D
C = 40
CP = 48
NB = NPAD // 128
R1T = ROWS // 16
R2T = ROWS // 32
STRIPE = NPAD // 16

_mesh = plsc.VectorSubcoreMesh(
    core_axis_name="c", subcore_axis_name="s", num_cores=2, num_subcores=16)


def _tc1_body(x_ref, w_ref, al_ref, ar_ref, feat_ref, el_ref, er_ref):
    f = jnp.dot(x_ref[...], w_ref[...], preferred_element_type=jnp.float32)
    for h in range(H):
        fh = f[:, h * D:(h + 1) * D]
        feat_ref[h] = fh
        el_ref[h] = jnp.sum(fh * al_ref[h][None, :], axis=1)
        er_ref[h] = jnp.sum(fh * ar_ref[h][None, :], axis=1)


def _dense1(x, W1, al1, ar1):
    return pl.pallas_call(
        _tc1_body,
        grid=(NB,),
        in_specs=[
            pl.BlockSpec((128, IN), lambda i: (i, 0)),
            pl.BlockSpec((IN, HD), lambda i: (0, 0)),
            pl.BlockSpec((H, D), lambda i: (0, 0)),
            pl.BlockSpec((H, D), lambda i: (0, 0)),
        ],
        out_specs=[
            pl.BlockSpec((H, 128, D), lambda i: (0, i, 0)),
            pl.BlockSpec((H, 128), lambda i: (0, i)),
            pl.BlockSpec((H, 128), lambda i: (0, i)),
        ],
        out_shape=[
            jax.ShapeDtypeStruct((H, NPAD, D), jnp.float32),
            jax.ShapeDtypeStruct((H, NPAD), jnp.float32),
            jax.ShapeDtypeStruct((H, NPAD), jnp.float32),
        ],
    )(x, W1, al1, ar1)


def _tc2_body(o1_ref, w2_ref, al2_ref, ar2_ref, f2_ref, elr_ref):
    h = pl.program_id(1)
    part = jnp.dot(o1_ref[0], w2_ref[...], preferred_element_type=jnp.float32)

    @pl.when(h == 0)
    def _():
        f2_ref[...] = part

    @pl.when(h != 0)
    def _():
        f2_ref[...] = f2_ref[...] + part

    @pl.when(h == H - 1)
    def _():
        f2 = f2_ref[...]
        elr_ref[0] = jnp.sum(f2 * al2_ref[0][None, :], axis=1)
        elr_ref[1] = jnp.sum(f2 * ar2_ref[0][None, :], axis=1)


def _dense2(out1H, W2p, al2p, ar2p):
    return pl.pallas_call(
        _tc2_body,
        grid=(NB, H),
        in_specs=[
            pl.BlockSpec((1, 128, D), lambda i, h: (h, i, 0)),
            pl.BlockSpec((D, CP), lambda i, h: (h, 0)),
            pl.BlockSpec((1, CP), lambda i, h: (0, 0)),
            pl.BlockSpec((1, CP), lambda i, h: (0, 0)),
        ],
        out_specs=[
            pl.BlockSpec((128, CP), lambda i, h: (i, 0)),
            pl.BlockSpec((2, 128), lambda i, h: (0, i)),
        ],
        out_shape=[
            jax.ShapeDtypeStruct((NPAD, CP), jnp.float32),
            jax.ShapeDtypeStruct((2, NPAD), jnp.float32),
        ],
    )(out1H, W2p, al2p, ar2p)


def _tc3_body(a_ref, s_ref, b_ref, o_ref):
    acc = a_ref[0] + a_ref[1]
    s = s_ref[0] + s_ref[1] + 1e-9
    o_ref[...] = acc / s[:, None] + b_ref[0][None, :]


def _dense3(accP, sP, b2p):
    return pl.pallas_call(
        _tc3_body,
        grid=(NB,),
        in_specs=[
            pl.BlockSpec((2, 128, CP), lambda i: (0, i, 0)),
            pl.BlockSpec((2, 128), lambda i: (0, i)),
            pl.BlockSpec((1, CP), lambda i: (0, 0)),
        ],
        out_specs=pl.BlockSpec((128, CP), lambda i: (i, 0)),
        out_shape=jax.ShapeDtypeStruct((NPAD, CP), jnp.float32),
    )(accP, sP, b2p)


def _sc1_body(feat_hbm, el_hbm, er_hbm, b_hbm, src_hbm, dst_hbm, out_hbm,
              srcb, dstb, el_v, er_v, b_v, exb, gidxb, rows, nodeb, sb, svec,
              acc_sh, s_sh, sem):
    cid = lax.axis_index("c")
    sid = lax.axis_index("s")

    pltpu.sync_copy(src_hbm.at[pl.ds(sid * R1T, R1T)], srcb)
    pltpu.sync_copy(dst_hbm.at[pl.ds(sid * R1T, R1T)], dstb)

    def _zero_svec(k, _):
        svec[pl.ds(k * 16, 16)] = jnp.zeros((16,), jnp.float32)
        return 0
    lax.fori_loop(0, STRIPE // 16, _zero_svec, 0)

    def head_body(i, _):
        h = cid * 4 + i
        pltpu.sync_copy(el_hbm.at[pl.ds(h * NPAD, NPAD)], el_v)
        pltpu.sync_copy(er_hbm.at[pl.ds(h * NPAD, NPAD)], er_v)
        pltpu.sync_copy(b_hbm.at[pl.ds(h * D, D)], b_v)

        def _zero_nodeb(j, _):
            for q in range(D // 16):
                nodeb[j, pl.ds(q * 16, 16)] = jnp.zeros((16,), jnp.float32)
            return 0
        lax.fori_loop(0, 128, _zero_nodeb, 0)
        for k in range(STRIPE // 128):
            pltpu.sync_copy(nodeb, acc_sh.at[pl.ds(sid * STRIPE + k * 128, 128)])
        pltpu.sync_copy(svec, s_sh.at[pl.ds(sid * STRIPE, STRIPE)])
        plsc.subcore_barrier()

        def chunk(r, _):
            for g in range(8):
                s16 = srcb[r, pl.ds(g * 16, 16)]
                d16 = dstb[r, pl.ds(g * 16, 16)]
                ea = plsc.load_gather(el_v, [s16])
                eb = plsc.load_gather(er_v, [d16])
                e = ea + eb
                e = jnp.where(e >= 0.0, e, 0.2 * e)
                ex = jnp.exp(e)
                eid = (sid * R1T + r) * 128 + g * 16 + lax.iota(jnp.int32, 16)
                ex = jnp.where(eid < E, ex, 0.0)
                exb[pl.ds(g * 16, 16)] = ex
                gidxb[pl.ds(g * 16, 16)] = s16 + h * NPAD
            pltpu.async_copy(feat_hbm.at[gidxb], rows, sem).wait()

            def scale(g, _):
                exv = exb[pl.ds(g * 16, 16)]
                for t in range(16):
                    es = exv[t]
                    j = g * 16 + t
                    for q in range(D // 16):
                        rows[j, pl.ds(q * 16, 16)] = (
                            rows[j, pl.ds(q * 16, 16)] * es)
                return 0
            lax.fori_loop(0, 8, scale, 0)
            pltpu.sync_copy(rows, acc_sh.at[dstb.at[r]], add=True)
            pltpu.sync_copy(exb, s_sh.at[dstb.at[r]], add=True)
            return 0
        lax.fori_loop(0, R1T, chunk, 0)
        plsc.subcore_barrier()

        for k in range(STRIPE // 128):
            r0 = sid * STRIPE + k * 128
            pltpu.sync_copy(acc_sh.at[pl.ds(r0, 128)], nodeb)
            pltpu.sync_copy(s_sh.at[pl.ds(r0, 128)], sb)

            def fin(g, _):
                invv = 1.0 / (sb[pl.ds(g * 16, 16)] + 1e-9)
                for t in range(16):
                    inv = invv[t]
                    j = g * 16 + t
                    for q in range(D // 16):
                        v = (nodeb[j, pl.ds(q * 16, 16)] * inv
                             + b_v[pl.ds(q * 16, 16)])
                        nodeb[j, pl.ds(q * 16, 16)] = jnp.where(
                            v > 0.0, v, jnp.exp(v) - 1.0)
                return 0
            lax.fori_loop(0, 8, fin, 0)
            pltpu.sync_copy(nodeb, out_hbm.at[pl.ds(h * NPAD + r0, 128)])
        plsc.subcore_barrier()
        return 0

    lax.fori_loop(0, 4, head_body, 0)


def _sc_layer1(featF, elF, erF, b1, srcR, dstR):
    fn = pl.kernel(
        _sc1_body,
        out_type=jax.ShapeDtypeStruct((H * NPAD, D), jnp.float32),
        mesh=_mesh,
        compiler_params=pltpu.CompilerParams(needs_layout_passes=False, use_tc_tiling_on_sc=False),
        scratch_types=[
            pltpu.VMEM((R1T, 128), jnp.int32),
            pltpu.VMEM((R1T, 128), jnp.int32),
            pltpu.VMEM((NPAD,), jnp.float32),
            pltpu.VMEM((NPAD,), jnp.float32),
            pltpu.VMEM((D,), jnp.float32),
            pltpu.VMEM((128,), jnp.float32),
            pltpu.VMEM((128,), jnp.int32),
            pltpu.VMEM((128, D), jnp.float32),
            pltpu.VMEM((128, D), jnp.float32),
            pltpu.VMEM((128,), jnp.float32),
            pltpu.VMEM((STRIPE,), jnp.float32),
            pltpu.VMEM_SHARED((NPAD, D), jnp.float32),
            pltpu.VMEM_SHARED((NPAD,), jnp.float32),
            pltpu.SemaphoreType.DMA,
        ],
    )
    return fn(featF, elF, erF, b1, srcR, dstR)


def _sc2_body(feat_hbm, el_hbm, er_hbm, src_hbm, dst_hbm, acc_out, s_out,
              srcb, dstb, el_v, er_v, exb, rows, nodeb, svec,
              acc_sh, s_sh, sem):
    cid = lax.axis_index("c")
    sid = lax.axis_index("s")
    wid = cid * 16 + sid

    pltpu.sync_copy(src_hbm.at[pl.ds(wid * R2T, R2T)], srcb)
    pltpu.sync_copy(dst_hbm.at[pl.ds(wid * R2T, R2T)], dstb)
    pltpu.sync_copy(el_hbm, el_v)
    pltpu.sync_copy(er_hbm, er_v)

    def _zero_nodeb(j, _):
        for q in range(CP // 16):
            nodeb[j, pl.ds(q * 16, 16)] = jnp.zeros((16,), jnp.float32)
        return 0
    lax.fori_loop(0, 128, _zero_nodeb, 0)

    def _zero_svec(k, _):
        svec[pl.ds(k * 16, 16)] = jnp.zeros((16,), jnp.float32)
        return 0
    lax.fori_loop(0, STRIPE // 16, _zero_svec, 0)

    for k in range(STRIPE // 128):
        pltpu.sync_copy(nodeb, acc_sh.at[pl.ds(sid * STRIPE + k * 128, 128)])
    pltpu.sync_copy(svec, s_sh.at[pl.ds(sid * STRIPE, STRIPE)])
    plsc.subcore_barrier()

    def chunk(r, _):
        for g in range(8):
            s16 = srcb[r, pl.ds(g * 16, 16)]
            d16 = dstb[r, pl.ds(g * 16, 16)]
            ea = plsc.load_gather(el_v, [s16])
            eb = plsc.load_gather(er_v, [d16])
            e = ea + eb
            e = jnp.where(e >= 0.0, e, 0.2 * e)
            ex = jnp.exp(e)
            eid = (wid * R2T + r) * 128 + g * 16 + lax.iota(jnp.int32, 16)
            ex = jnp.where(eid < E, ex, 0.0)
            exb[pl.ds(g * 16, 16)] = ex
        pltpu.async_copy(feat_hbm.at[srcb.at[r]], rows, sem).wait()

        def scale(g, _):
            exv = exb[pl.ds(g * 16, 16)]
            for t in range(16):
                es = exv[t]
                j = g * 16 + t
                for q in range(CP // 16):
                    rows[j, pl.ds(q * 16, 16)] = rows[j, pl.ds(q * 16, 16)] * es
            return 0
        lax.fori_loop(0, 8, scale, 0)
        pltpu.sync_copy(rows, acc_sh.at[dstb.at[r]], add=True)
        pltpu.sync_copy(exb, s_sh.at[dstb.at[r]], add=True)
        return 0
    lax.fori_loop(0, R2T, chunk, 0)
    plsc.subcore_barrier()

    for k in range(STRIPE // 128):
        r0 = sid * STRIPE + k * 128
        pltpu.sync_copy(acc_sh.at[pl.ds(r0, 128)], nodeb)
        pltpu.sync_copy(nodeb, acc_out.at[pl.ds(cid * NPAD + r0, 128)])
    pltpu.sync_copy(s_sh.at[pl.ds(sid * STRIPE, STRIPE)], svec)
    pltpu.sync_copy(svec, s_out.at[pl.ds(cid * NPAD + sid * STRIPE, STRIPE)])


def _sc_layer2(feat2, el2, er2, srcR, dstR):
    fn = pl.kernel(
        _sc2_body,
        out_type=(
            jax.ShapeDtypeStruct((2 * NPAD, CP), jnp.float32),
            jax.ShapeDtypeStruct((2 * NPAD,), jnp.float32),
        ),
        mesh=_mesh,
        compiler_params=pltpu.CompilerParams(needs_layout_passes=False, use_tc_tiling_on_sc=False),
        scratch_types=[
            pltpu.VMEM((R2T, 128), jnp.int32),
            pltpu.VMEM((R2T, 128), jnp.int32),
            pltpu.VMEM((NPAD,), jnp.float32),
            pltpu.VMEM((NPAD,), jnp.float32),
            pltpu.VMEM((128,), jnp.float32),
            pltpu.VMEM((128, CP), jnp.float32),
            pltpu.VMEM((128, CP), jnp.float32),
            pltpu.VMEM((STRIPE,), jnp.float32),
            pltpu.VMEM_SHARED((NPAD, CP), jnp.float32),
            pltpu.VMEM_SHARED((NPAD,), jnp.float32),
            pltpu.SemaphoreType.DMA,
        ],
    )
    return fn(feat2, el2, er2, srcR, dstR)


def kernel(features, edge_index, W1, al1, ar1, b1, W2, al2, ar2, b2):
    xp = jnp.pad(features, ((0, NPAD - N), (0, 0)))
    featH, elH, erH = _dense1(xp, W1, al1, ar1)

    src = edge_index[0]
    dst = edge_index[1]
    srcR = jnp.pad(src, (0, EPAD - E)).reshape(ROWS, 128)
    dstR = jnp.pad(dst, (0, EPAD - E)).reshape(ROWS, 128)

    out1F = _sc_layer1(featH.reshape(H * NPAD, D), elH.reshape(-1),
                       erH.reshape(-1), b1, srcR, dstR)
    out1H = out1F.reshape(H, NPAD, D)

    W2p = jnp.pad(W2, ((0, 0), (0, CP - C)))
    al2p = jnp.pad(al2.reshape(1, C), ((0, 0), (0, CP - C)))
    ar2p = jnp.pad(ar2.reshape(1, C), ((0, 0), (0, CP - C)))
    feat2, elr2 = _dense2(out1H, W2p, al2p, ar2p)

    acc2F, s2F = _sc_layer2(feat2, elr2[0], elr2[1], srcR, dstR)

    b2p = jnp.pad(b2.reshape(1, C), ((0, 0), (0, CP - C)))
    out = _dense3(acc2F.reshape(2, NPAD, CP), s2F.reshape(2, NPAD), b2p)
    return out[:N, :C]


# speedup vs baseline: 16.0766x; 1.2761x over previous
import functools

import jax
import jax.numpy as jnp
from jax import lax
from jax.experimental import pallas as pl
from jax.experimental.pallas import tpu as pltpu
from jax.experimental.pallas import tpu_sc as plsc

N = 10000
NPAD = 10240
E = 320000
ROWS = 2560
EPAD = ROWS * 128
IN = 128
H = 8
D = 64
HD = H * D
C = 40
CP = 48
NB = NPAD // 128
R1T = ROWS // 16
R2T = ROWS // 32
STRIPE = NPAD // 16

_mesh = plsc.VectorSubcoreMesh(
    core_axis_name="c", subcore_axis_name="s", num_cores=2, num_subcores=16)


def _tc1_body(x_ref, w_ref, al_ref, ar_ref, feat_ref, el_ref, er_ref):
    f = jnp.dot(x_ref[...], w_ref[...], preferred_element_type=jnp.float32)
    for h in range(H):
        fh = f[:, h * D:(h + 1) * D]
        feat_ref[h] = fh
        el_ref[h] = jnp.sum(fh * al_ref[h][None, :], axis=1)
        er_ref[h] = jnp.sum(fh * ar_ref[h][None, :], axis=1)


def _dense1(x, W1, al1, ar1):
    return pl.pallas_call(
        _tc1_body,
        grid=(NB,),
        in_specs=[
            pl.BlockSpec((128, IN), lambda i: (i, 0)),
            pl.BlockSpec((IN, HD), lambda i: (0, 0)),
            pl.BlockSpec((H, D), lambda i: (0, 0)),
            pl.BlockSpec((H, D), lambda i: (0, 0)),
        ],
        out_specs=[
            pl.BlockSpec((H, 128, D), lambda i: (0, i, 0)),
            pl.BlockSpec((H, 128), lambda i: (0, i)),
            pl.BlockSpec((H, 128), lambda i: (0, i)),
        ],
        out_shape=[
            jax.ShapeDtypeStruct((H, NPAD, D), jnp.float32),
            jax.ShapeDtypeStruct((H, NPAD), jnp.float32),
            jax.ShapeDtypeStruct((H, NPAD), jnp.float32),
        ],
    )(x, W1, al1, ar1)


def _tc2_body(o1_ref, w2_ref, al2_ref, ar2_ref, f2_ref, elr_ref):
    h = pl.program_id(1)
    part = jnp.dot(o1_ref[0], w2_ref[...], preferred_element_type=jnp.float32)

    @pl.when(h == 0)
    def _():
        f2_ref[...] = part

    @pl.when(h != 0)
    def _():
        f2_ref[...] = f2_ref[...] + part

    @pl.when(h == H - 1)
    def _():
        f2 = f2_ref[...]
        elr_ref[0] = jnp.sum(f2 * al2_ref[0][None, :], axis=1)
        elr_ref[1] = jnp.sum(f2 * ar2_ref[0][None, :], axis=1)


def _dense2(out1H, W2p, al2p, ar2p):
    return pl.pallas_call(
        _tc2_body,
        grid=(NB, H),
        in_specs=[
            pl.BlockSpec((1, 128, D), lambda i, h: (h, i, 0)),
            pl.BlockSpec((D, CP), lambda i, h: (h, 0)),
            pl.BlockSpec((1, CP), lambda i, h: (0, 0)),
            pl.BlockSpec((1, CP), lambda i, h: (0, 0)),
        ],
        out_specs=[
            pl.BlockSpec((128, CP), lambda i, h: (i, 0)),
            pl.BlockSpec((2, 128), lambda i, h: (0, i)),
        ],
        out_shape=[
            jax.ShapeDtypeStruct((NPAD, CP), jnp.float32),
            jax.ShapeDtypeStruct((2, NPAD), jnp.float32),
        ],
    )(out1H, W2p, al2p, ar2p)


def _tc3_body(a_ref, s_ref, b_ref, o_ref):
    acc = a_ref[0] + a_ref[1]
    s = s_ref[0] + s_ref[1] + 1e-9
    o_ref[...] = acc / s[:, None] + b_ref[0][None, :]


def _dense3(accP, sP, b2p):
    return pl.pallas_call(
        _tc3_body,
        grid=(NB,),
        in_specs=[
            pl.BlockSpec((2, 128, CP), lambda i: (0, i, 0)),
            pl.BlockSpec((2, 128), lambda i: (0, i)),
            pl.BlockSpec((1, CP), lambda i: (0, 0)),
        ],
        out_specs=pl.BlockSpec((128, CP), lambda i: (i, 0)),
        out_shape=jax.ShapeDtypeStruct((NPAD, CP), jnp.float32),
    )(accP, sP, b2p)


def _sc1_body(feat_hbm, el_hbm, er_hbm, b_hbm, src_hbm, dst_hbm, out_hbm,
              srcb, dstc, el_v, er_v, b_v, exb0, exb1, exb2, exb3,
              rows0, rows1, rows2, rows3,
              nodeb, sb, svec, acc_sh, s_sh,
              sem0, sem1, sem2, sem3, dsem0, dsem1, dsem2, dsem3):
    cid = lax.axis_index("c")
    sid = lax.axis_index("s")
    rings = (rows0, rows1, rows2, rows3)
    exbufs = (exb0, exb1, exb2, exb3)
    sems = (sem0, sem1, sem2, sem3)
    dsems = (dsem0, dsem1, dsem2, dsem3)

    pltpu.sync_copy(src_hbm.at[pl.ds(sid * R1T, R1T)], srcb)

    def _zero_svec(k, _):
        svec[pl.ds(k * 16, 16)] = jnp.zeros((16,), jnp.float32)
        return 0
    lax.fori_loop(0, STRIPE // 16, _zero_svec, 0)

    def head_body(i, _):
        h = cid * 4 + i
        featH = feat_hbm.at[pl.ds(h * NPAD, NPAD)]
        pltpu.sync_copy(el_hbm.at[pl.ds(h * NPAD, NPAD)], el_v)
        pltpu.sync_copy(er_hbm.at[pl.ds(h * NPAD, NPAD)], er_v)
        pltpu.sync_copy(b_hbm.at[pl.ds(h * D, D)], b_v)

        def _zero_nodeb(j, _):
            for q in range(D // 16):
                nodeb[j, pl.ds(q * 16, 16)] = jnp.zeros((16,), jnp.float32)
            return 0
        lax.fori_loop(0, 128, _zero_nodeb, 0)
        for k in range(STRIPE // 128):
            pltpu.sync_copy(nodeb, acc_sh.at[pl.ds(sid * STRIPE + k * 128, 128)])
        pltpu.sync_copy(svec, s_sh.at[pl.ds(sid * STRIPE, STRIPE)])
        plsc.subcore_barrier()

        for b in range(4):
            pltpu.async_copy(dst_hbm.at[sid * R1T + b], dstc.at[b], dsems[b])
            pltpu.async_copy(featH.at[srcb.at[b]], rings[b], sems[b])

        def pipe(i4, _):
            for b in range(4):
                r = i4 * 4 + b
                buf = rings[b]
                exb = exbufs[b]
                pltpu.make_async_copy(dst_hbm.at[sid * R1T + r], dstc.at[b],
                                      dsems[b]).wait()
                for g in range(8):
                    s16 = srcb[r, pl.ds(g * 16, 16)]
                    d16 = dstc[b, pl.ds(g * 16, 16)]
                    ea = plsc.load_gather(el_v, [s16])
                    eb = plsc.load_gather(er_v, [d16])
                    e = ea + eb
                    e = jnp.where(e >= 0.0, e, 0.2 * e)
                    ex = jnp.exp(e)
                    eid = ((sid * R1T + r) * 128 + g * 16
                           + lax.iota(jnp.int32, 16))
                    ex = jnp.where(eid < E, ex, 0.0)
                    exb[pl.ds(g * 16, 16)] = ex
                pltpu.make_async_copy(featH.at[srcb.at[r]], buf,
                                      sems[b]).wait()

                def scale(g, _):
                    exv = exb[pl.ds(g * 16, 16)]
                    for t in range(16):
                        es = exv[t]
                        j = g * 16 + t
                        for q in range(D // 16):
                            buf[j, pl.ds(q * 16, 16)] = (
                                buf[j, pl.ds(q * 16, 16)] * es)
                    return 0
                lax.fori_loop(0, 8, scale, 0)
                pltpu.sync_copy(buf, acc_sh.at[dstc.at[b]], add=True)
                pltpu.sync_copy(exb, s_sh.at[dstc.at[b]], add=True)

                @pl.when(r + 4 < R1T)
                def _():
                    pltpu.async_copy(dst_hbm.at[sid * R1T + r + 4],
                                     dstc.at[b], dsems[b])
                    pltpu.async_copy(featH.at[srcb.at[r + 4]], buf, sems[b])
            return 0
        lax.fori_loop(0, R1T // 4, pipe, 0)
        plsc.subcore_barrier()

        for k in range(STRIPE // 128):
            r0 = sid * STRIPE + k * 128
            pltpu.sync_copy(acc_sh.at[pl.ds(r0, 128)], nodeb)
            pltpu.sync_copy(s_sh.at[pl.ds(r0, 128)], sb)

            def fin(g, _):
                invv = 1.0 / (sb[pl.ds(g * 16, 16)] + 1e-9)
                for t in range(16):
                    inv = invv[t]
                    j = g * 16 + t
                    for q in range(D // 16):
                        v = (nodeb[j, pl.ds(q * 16, 16)] * inv
                             + b_v[pl.ds(q * 16, 16)])
                        nodeb[j, pl.ds(q * 16, 16)] = jnp.where(
                            v > 0.0, v, jnp.exp(v) - 1.0)
                return 0
            lax.fori_loop(0, 8, fin, 0)
            pltpu.sync_copy(nodeb, out_hbm.at[pl.ds(h * NPAD + r0, 128)])
        plsc.subcore_barrier()
        return 0

    lax.fori_loop(0, 4, head_body, 0)


def _sc_layer1(featF, elF, erF, b1, srcR, dstR):
    fn = pl.kernel(
        _sc1_body,
        out_type=jax.ShapeDtypeStruct((H * NPAD, D), jnp.float32),
        mesh=_mesh,
        compiler_params=pltpu.CompilerParams(needs_layout_passes=False, use_tc_tiling_on_sc=False),
        scratch_types=[
            pltpu.VMEM((R1T, 128), jnp.int32),
            pltpu.VMEM((4, 128), jnp.int32),
            pltpu.VMEM((NPAD,), jnp.float32),
            pltpu.VMEM((NPAD,), jnp.float32),
            pltpu.VMEM((D,), jnp.float32),
            pltpu.VMEM((128,), jnp.float32),
            pltpu.VMEM((128,), jnp.float32),
            pltpu.VMEM((128,), jnp.float32),
            pltpu.VMEM((128,), jnp.float32),
            pltpu.VMEM((128, D), jnp.float32),
            pltpu.VMEM((128, D), jnp.float32),
            pltpu.VMEM((128, D), jnp.float32),
            pltpu.VMEM((128, D), jnp.float32),
            pltpu.VMEM((128, D), jnp.float32),
            pltpu.VMEM((128,), jnp.float32),
            pltpu.VMEM((STRIPE,), jnp.float32),
            pltpu.VMEM_SHARED((NPAD, D), jnp.float32),
            pltpu.VMEM_SHARED((NPAD,), jnp.float32),
            pltpu.SemaphoreType.DMA,
            pltpu.SemaphoreType.DMA,
            pltpu.SemaphoreType.DMA,
            pltpu.SemaphoreType.DMA,
            pltpu.SemaphoreType.DMA,
            pltpu.SemaphoreType.DMA,
            pltpu.SemaphoreType.DMA,
            pltpu.SemaphoreType.DMA,
        ],
    )
    return fn(featF, elF, erF, b1, srcR, dstR)


NH2 = NPAD // 2
STRIPE2 = NH2 // 16


def _sc2_body(feat_hbm, el_hbm, er_hbm, b_hbm, src_hbm, dst_hbm, out_hbm,
              srcb, dstc, el_v, er_v, b_v, exb0, exb1, exb2, exb3,
              rows0, rows1, rows2, rows3, nodeb, sb, acc_sh, s_sh,
              sem0, sem1, sem2, sem3, dsem0, dsem1, dsem2, dsem3):
    cid = lax.axis_index("c")
    sid = lax.axis_index("s")
    rings = (rows0, rows1, rows2, rows3)
    exbufs = (exb0, exb1, exb2, exb3)
    sems = (sem0, sem1, sem2, sem3)
    dsems = (dsem0, dsem1, dsem2, dsem3)
    lo = cid * NH2

    pltpu.sync_copy(src_hbm.at[pl.ds(sid * R1T, R1T)], srcb)
    pltpu.sync_copy(el_hbm, el_v)
    pltpu.sync_copy(er_hbm, er_v)
    pltpu.sync_copy(b_hbm, b_v)

    def _zero_nodeb(j, _):
        for q in range(CP // 16):
            nodeb[j, pl.ds(q * 16, 16)] = jnp.zeros((16,), jnp.float32)
        return 0
    lax.fori_loop(0, STRIPE2, _zero_nodeb, 0)

    def _zero_sb(k, _):
        sb[pl.ds(k * 16, 16)] = jnp.zeros((16,), jnp.float32)
        return 0
    lax.fori_loop(0, STRIPE2 // 16, _zero_sb, 0)

    pltpu.sync_copy(nodeb, acc_sh.at[pl.ds(sid * STRIPE2, STRIPE2)])
    pltpu.sync_copy(sb, s_sh.at[pl.ds(sid * STRIPE2, STRIPE2)])
    plsc.subcore_barrier()

    for b in range(4):
        pltpu.async_copy(dst_hbm.at[sid * R1T + b], dstc.at[b], dsems[b])
        pltpu.async_copy(feat_hbm.at[srcb.at[b]], rings[b], sems[b])

    def pipe(i4, _):
        for b in range(4):
            r = i4 * 4 + b
            buf = rings[b]
            exb = exbufs[b]
            pltpu.make_async_copy(dst_hbm.at[sid * R1T + r], dstc.at[b],
                                  dsems[b]).wait()
            for g in range(8):
                s16 = srcb[r, pl.ds(g * 16, 16)]
                d16 = dstc[b, pl.ds(g * 16, 16)]
                ea = plsc.load_gather(el_v, [s16])
                eb = plsc.load_gather(er_v, [d16])
                e = ea + eb
                e = jnp.where(e >= 0.0, e, 0.2 * e)
                ex = jnp.exp(e)
                eid = (sid * R1T + r) * 128 + g * 16 + lax.iota(jnp.int32, 16)
                dl = d16 - lo
                mine = (dl >= 0) & (dl < NH2)
                ex = jnp.where((eid < E) & mine, ex, 0.0)
                exb[pl.ds(g * 16, 16)] = ex
                dstc[b, pl.ds(g * 16, 16)] = jnp.where(mine, dl, 0)
            pltpu.make_async_copy(feat_hbm.at[srcb.at[r]], buf, sems[b]).wait()

            def scale(g, _):
                exv = exb[pl.ds(g * 16, 16)]
                for t in range(16):
                    es = exv[t]
                    j = g * 16 + t
                    for q in range(CP // 16):
                        buf[j, pl.ds(q * 16, 16)] = buf[j, pl.ds(q * 16, 16)] * es
                return 0
            lax.fori_loop(0, 8, scale, 0)
            pltpu.sync_copy(buf, acc_sh.at[dstc.at[b]], add=True)
            pltpu.sync_copy(exb, s_sh.at[dstc.at[b]], add=True)

            @pl.when(r + 4 < R1T)
            def _():
                pltpu.async_copy(dst_hbm.at[sid * R1T + r + 4],
                                 dstc.at[b], dsems[b])
                pltpu.async_copy(feat_hbm.at[srcb.at[r + 4]], buf, sems[b])
        return 0
    lax.fori_loop(0, R1T // 4, pipe, 0)
    plsc.subcore_barrier()

    r0 = sid * STRIPE2
    pltpu.sync_copy(acc_sh.at[pl.ds(r0, STRIPE2)], nodeb)
    pltpu.sync_copy(s_sh.at[pl.ds(r0, STRIPE2)], sb)

    def fin(g, _):
        invv = 1.0 / (sb[pl.ds(g * 16, 16)] + 1e-9)
        for t in range(16):
            inv = invv[t]
            j = g * 16 + t
            for q in range(CP // 16):
                nodeb[j, pl.ds(q * 16, 16)] = (
                    nodeb[j, pl.ds(q * 16, 16)] * inv + b_v[pl.ds(q * 16, 16)])
        return 0
    lax.fori_loop(0, STRIPE2 // 16, fin, 0)
    pltpu.sync_copy(nodeb, out_hbm.at[pl.ds(lo + r0, STRIPE2)])


def _sc_layer2(feat2, el2, er2, b2p, srcR, dstR):
    fn = pl.kernel(
        _sc2_body,
        out_type=jax.ShapeDtypeStruct((NPAD, CP), jnp.float32),
        mesh=_mesh,
        compiler_params=pltpu.CompilerParams(needs_layout_passes=False, use_tc_tiling_on_sc=False),
        scratch_types=[
            pltpu.VMEM((R1T, 128), jnp.int32),
            pltpu.VMEM((4, 128), jnp.int32),
            pltpu.VMEM((NPAD,), jnp.float32),
            pltpu.VMEM((NPAD,), jnp.float32),
            pltpu.VMEM((CP,), jnp.float32),
            pltpu.VMEM((128,), jnp.float32),
            pltpu.VMEM((128,), jnp.float32),
            pltpu.VMEM((128,), jnp.float32),
            pltpu.VMEM((128,), jnp.float32),
            pltpu.VMEM((128, CP), jnp.float32),
            pltpu.VMEM((128, CP), jnp.float32),
            pltpu.VMEM((128, CP), jnp.float32),
            pltpu.VMEM((128, CP), jnp.float32),
            pltpu.VMEM((STRIPE2, CP), jnp.float32),
            pltpu.VMEM((STRIPE2,), jnp.float32),
            pltpu.VMEM_SHARED((NH2, CP), jnp.float32),
            pltpu.VMEM_SHARED((NH2,), jnp.float32),
            pltpu.SemaphoreType.DMA,
            pltpu.SemaphoreType.DMA,
            pltpu.SemaphoreType.DMA,
            pltpu.SemaphoreType.DMA,
            pltpu.SemaphoreType.DMA,
            pltpu.SemaphoreType.DMA,
            pltpu.SemaphoreType.DMA,
            pltpu.SemaphoreType.DMA,
        ],
    )
    return fn(feat2, el2, er2, b2p, srcR, dstR)


def kernel(features, edge_index, W1, al1, ar1, b1, W2, al2, ar2, b2):
    xp = jnp.pad(features, ((0, NPAD - N), (0, 0)))
    featH, elH, erH = _dense1(xp, W1, al1, ar1)

    src = edge_index[0]
    dst = edge_index[1]
    srcR = jnp.pad(src, (0, EPAD - E)).reshape(ROWS, 128)
    dstR = jnp.pad(dst, (0, EPAD - E)).reshape(ROWS, 128)

    out1F = _sc_layer1(featH.reshape(H * NPAD, D), elH.reshape(-1),
                       erH.reshape(-1), b1, srcR, dstR)
    out1H = out1F.reshape(H, NPAD, D)

    W2p = jnp.pad(W2, ((0, 0), (0, CP - C)))
    al2p = jnp.pad(al2.reshape(1, C), ((0, 0), (0, CP - C)))
    ar2p = jnp.pad(ar2.reshape(1, C), ((0, 0), (0, CP - C)))
    feat2, elr2 = _dense2(out1H, W2p, al2p, ar2p)

    b2p = jnp.pad(b2.reshape(C), (0, CP - C))
    out = _sc_layer2(feat2, elr2[0], elr2[1], b2p, srcR, dstR)
    return out[:N, :C]


# speedup vs baseline: 17.6541x; 1.0981x over previous
import functools

import jax
import jax.numpy as jnp
from jax import lax
from jax.experimental import pallas as pl
from jax.experimental.pallas import tpu as pltpu
from jax.experimental.pallas import tpu_sc as plsc

N = 10000
NPAD = 10240
E = 320000
ROWS = 2560
EPAD = ROWS * 128
IN = 128
H = 8
D = 64
HD = H * D
C = 40
CP = 48
NB = NPAD // 128
R1T = ROWS // 16
R2T = ROWS // 32
STRIPE = NPAD // 16

_mesh = plsc.VectorSubcoreMesh(
    core_axis_name="c", subcore_axis_name="s", num_cores=2, num_subcores=16)


def _tc1_body(x_ref, w_ref, al_ref, ar_ref, feat_ref, el_ref, er_ref):
    f = jnp.dot(x_ref[...], w_ref[...], preferred_element_type=jnp.float32)
    for h in range(H):
        fh = f[:, h * D:(h + 1) * D]
        feat_ref[h] = fh
        el_ref[h] = jnp.sum(fh * al_ref[h][None, :], axis=1)
        er_ref[h] = jnp.sum(fh * ar_ref[h][None, :], axis=1)


def _dense1(x, W1, al1, ar1):
    return pl.pallas_call(
        _tc1_body,
        grid=(NB,),
        in_specs=[
            pl.BlockSpec((128, IN), lambda i: (i, 0)),
            pl.BlockSpec((IN, HD), lambda i: (0, 0)),
            pl.BlockSpec((H, D), lambda i: (0, 0)),
            pl.BlockSpec((H, D), lambda i: (0, 0)),
        ],
        out_specs=[
            pl.BlockSpec((H, 128, D), lambda i: (0, i, 0)),
            pl.BlockSpec((H, 128), lambda i: (0, i)),
            pl.BlockSpec((H, 128), lambda i: (0, i)),
        ],
        out_shape=[
            jax.ShapeDtypeStruct((H, NPAD, D), jnp.float32),
            jax.ShapeDtypeStruct((H, NPAD), jnp.float32),
            jax.ShapeDtypeStruct((H, NPAD), jnp.float32),
        ],
    )(x, W1, al1, ar1)


def _tc2_body(*refs):
    o1_refs = refs[:H]
    w2_ref, al2_ref, ar2_ref, f2_ref, elr_ref = refs[H:]
    x = jnp.concatenate([o1_refs[h][...] for h in range(H)], axis=1)
    f2 = jnp.dot(x, w2_ref[...], preferred_element_type=jnp.float32)
    f2_ref[...] = f2
    elr_ref[0] = jnp.sum(f2 * al2_ref[0][None, :], axis=1)
    elr_ref[1] = jnp.sum(f2 * ar2_ref[0][None, :], axis=1)


def _dense2(out1F, W2p, al2p, ar2p):
    def _spec(h):
        return pl.BlockSpec((128, D), lambda i, h=h: (h * NB + i, 0))
    return pl.pallas_call(
        _tc2_body,
        grid=(NB,),
        in_specs=[_spec(h) for h in range(H)] + [
            pl.BlockSpec((HD, CP), lambda i: (0, 0)),
            pl.BlockSpec((1, CP), lambda i: (0, 0)),
            pl.BlockSpec((1, CP), lambda i: (0, 0)),
        ],
        out_specs=[
            pl.BlockSpec((128, CP), lambda i: (i, 0)),
            pl.BlockSpec((2, 128), lambda i: (0, i)),
        ],
        out_shape=[
            jax.ShapeDtypeStruct((NPAD, CP), jnp.float32),
            jax.ShapeDtypeStruct((2, NPAD), jnp.float32),
        ],
    )(*([out1F] * H), W2p, al2p, ar2p)


def _tc3_body(a_ref, s_ref, b_ref, o_ref):
    acc = a_ref[0] + a_ref[1]
    s = s_ref[0] + s_ref[1] + 1e-9
    o_ref[...] = acc / s[:, None] + b_ref[0][None, :]


def _dense3(accP, sP, b2p):
    return pl.pallas_call(
        _tc3_body,
        grid=(NB,),
        in_specs=[
            pl.BlockSpec((2, 128, CP), lambda i: (0, i, 0)),
            pl.BlockSpec((2, 128), lambda i: (0, i)),
            pl.BlockSpec((1, CP), lambda i: (0, 0)),
        ],
        out_specs=pl.BlockSpec((128, CP), lambda i: (i, 0)),
        out_shape=jax.ShapeDtypeStruct((NPAD, CP), jnp.float32),
    )(accP, sP, b2p)


def _sc1_body(feat_hbm, el_hbm, er_hbm, b_hbm, src_hbm, dst_hbm, out_hbm,
              srcb, dstc, el_v, er_v, b_v, exb0, exb1, exb2, exb3,
              rows0, rows1, rows2, rows3,
              nodeb, sb, svec, acc_sh, s_sh,
              sem0, sem1, sem2, sem3, dsem0, dsem1, dsem2, dsem3):
    cid = lax.axis_index("c")
    sid = lax.axis_index("s")
    rings = (rows0, rows1, rows2, rows3)
    exbufs = (exb0, exb1, exb2, exb3)
    sems = (sem0, sem1, sem2, sem3)
    dsems = (dsem0, dsem1, dsem2, dsem3)

    pltpu.sync_copy(src_hbm.at[pl.ds(sid * R1T, R1T)], srcb)

    def _zero_svec(k, _):
        svec[pl.ds(k * 16, 16)] = jnp.zeros((16,), jnp.float32)
        return 0
    lax.fori_loop(0, STRIPE // 16, _zero_svec, 0)

    def head_body(i, _):
        h = cid * 4 + i
        featH = feat_hbm.at[pl.ds(h * NPAD, NPAD)]
        pltpu.sync_copy(el_hbm.at[pl.ds(h * NPAD, NPAD)], el_v)
        pltpu.sync_copy(er_hbm.at[pl.ds(h * NPAD, NPAD)], er_v)
        pltpu.sync_copy(b_hbm.at[pl.ds(h * D, D)], b_v)

        def _zero_nodeb(j, _):
            for q in range(D // 16):
                nodeb[j, pl.ds(q * 16, 16)] = jnp.zeros((16,), jnp.float32)
            return 0
        lax.fori_loop(0, 128, _zero_nodeb, 0)
        for k in range(STRIPE // 128):
            pltpu.sync_copy(nodeb, acc_sh.at[pl.ds(sid * STRIPE + k * 128, 128)])
        pltpu.sync_copy(svec, s_sh.at[pl.ds(sid * STRIPE, STRIPE)])
        plsc.subcore_barrier()

        for b in range(4):
            pltpu.async_copy(dst_hbm.at[sid * R1T + b], dstc.at[b], dsems[b])
            pltpu.async_copy(featH.at[srcb.at[b]], rings[b], sems[b])

        def pipe(i4, _):
            for b in range(4):
                r = i4 * 4 + b
                buf = rings[b]
                exb = exbufs[b]
                pltpu.make_async_copy(dst_hbm.at[sid * R1T + r], dstc.at[b],
                                      dsems[b]).wait()
                for g in range(8):
                    s16 = srcb[r, pl.ds(g * 16, 16)]
                    d16 = dstc[b, pl.ds(g * 16, 16)]
                    ea = plsc.load_gather(el_v, [s16])
                    eb = plsc.load_gather(er_v, [d16])
                    e = ea + eb
                    e = jnp.where(e >= 0.0, e, 0.2 * e)
                    ex = jnp.exp(e)
                    eid = ((sid * R1T + r) * 128 + g * 16
                           + lax.iota(jnp.int32, 16))
                    ex = jnp.where(eid < E, ex, 0.0)
                    exb[pl.ds(g * 16, 16)] = ex
                pltpu.make_async_copy(featH.at[srcb.at[r]], buf,
                                      sems[b]).wait()

                def scale(g, _):
                    exv = exb[pl.ds(g * 16, 16)]
                    for t in range(16):
                        es = exv[t]
                        j = g * 16 + t
                        for q in range(D // 16):
                            buf[j, pl.ds(q * 16, 16)] = (
                                buf[j, pl.ds(q * 16, 16)] * es)
                    return 0
                lax.fori_loop(0, 8, scale, 0)
                pltpu.sync_copy(buf, acc_sh.at[dstc.at[b]], add=True)
                pltpu.sync_copy(exb, s_sh.at[dstc.at[b]], add=True)

                @pl.when(r + 4 < R1T)
                def _():
                    pltpu.async_copy(dst_hbm.at[sid * R1T + r + 4],
                                     dstc.at[b], dsems[b])
                    pltpu.async_copy(featH.at[srcb.at[r + 4]], buf, sems[b])
            return 0
        lax.fori_loop(0, R1T // 4, pipe, 0)
        plsc.subcore_barrier()

        for k in range(STRIPE // 128):
            r0 = sid * STRIPE + k * 128
            pltpu.sync_copy(acc_sh.at[pl.ds(r0, 128)], nodeb)
            pltpu.sync_copy(s_sh.at[pl.ds(r0, 128)], sb)

            def fin(g, _):
                invv = 1.0 / (sb[pl.ds(g * 16, 16)] + 1e-9)
                for t in range(16):
                    inv = invv[t]
                    j = g * 16 + t
                    for q in range(D // 16):
                        v = (nodeb[j, pl.ds(q * 16, 16)] * inv
                             + b_v[pl.ds(q * 16, 16)])
                        nodeb[j, pl.ds(q * 16, 16)] = jnp.where(
                            v > 0.0, v, jnp.exp(v) - 1.0)
                return 0
            lax.fori_loop(0, 8, fin, 0)
            pltpu.sync_copy(nodeb, out_hbm.at[pl.ds(h * NPAD + r0, 128)])
        plsc.subcore_barrier()
        return 0

    lax.fori_loop(0, 4, head_body, 0)


def _sc_layer1(featF, elF, erF, b1, srcR, dstR):
    fn = pl.kernel(
        _sc1_body,
        out_type=jax.ShapeDtypeStruct((H * NPAD, D), jnp.float32),
        mesh=_mesh,
        compiler_params=pltpu.CompilerParams(needs_layout_passes=False, use_tc_tiling_on_sc=False),
        scratch_types=[
            pltpu.VMEM((R1T, 128), jnp.int32),
            pltpu.VMEM((4, 128), jnp.int32),
            pltpu.VMEM((NPAD,), jnp.float32),
            pltpu.VMEM((NPAD,), jnp.float32),
            pltpu.VMEM((D,), jnp.float32),
            pltpu.VMEM((128,), jnp.float32),
            pltpu.VMEM((128,), jnp.float32),
            pltpu.VMEM((128,), jnp.float32),
            pltpu.VMEM((128,), jnp.float32),
            pltpu.VMEM((128, D), jnp.float32),
            pltpu.VMEM((128, D), jnp.float32),
            pltpu.VMEM((128, D), jnp.float32),
            pltpu.VMEM((128, D), jnp.float32),
            pltpu.VMEM((128, D), jnp.float32),
            pltpu.VMEM((128,), jnp.float32),
            pltpu.VMEM((STRIPE,), jnp.float32),
            pltpu.VMEM_SHARED((NPAD, D), jnp.float32),
            pltpu.VMEM_SHARED((NPAD,), jnp.float32),
            pltpu.SemaphoreType.DMA,
            pltpu.SemaphoreType.DMA,
            pltpu.SemaphoreType.DMA,
            pltpu.SemaphoreType.DMA,
            pltpu.SemaphoreType.DMA,
            pltpu.SemaphoreType.DMA,
            pltpu.SemaphoreType.DMA,
            pltpu.SemaphoreType.DMA,
        ],
    )
    return fn(featF, elF, erF, b1, srcR, dstR)


NH2 = NPAD // 2
STRIPE2 = NH2 // 16


def _sc2_body(feat_hbm, el_hbm, er_hbm, b_hbm, src_hbm, dst_hbm, out_hbm,
              srcb, dstc, el_v, er_v, b_v, exb0, exb1, exb2, exb3,
              rows0, rows1, rows2, rows3, nodeb, sb, acc_sh, s_sh,
              sem0, sem1, sem2, sem3, dsem0, dsem1, dsem2, dsem3):
    cid = lax.axis_index("c")
    sid = lax.axis_index("s")
    rings = (rows0, rows1, rows2, rows3)
    exbufs = (exb0, exb1, exb2, exb3)
    sems = (sem0, sem1, sem2, sem3)
    dsems = (dsem0, dsem1, dsem2, dsem3)
    lo = cid * NH2

    pltpu.sync_copy(src_hbm.at[pl.ds(sid * R1T, R1T)], srcb)
    pltpu.sync_copy(el_hbm, el_v)
    pltpu.sync_copy(er_hbm, er_v)
    pltpu.sync_copy(b_hbm, b_v)

    def _zero_nodeb(j, _):
        for q in range(CP // 16):
            nodeb[j, pl.ds(q * 16, 16)] = jnp.zeros((16,), jnp.float32)
        return 0
    lax.fori_loop(0, STRIPE2, _zero_nodeb, 0)

    def _zero_sb(k, _):
        sb[pl.ds(k * 16, 16)] = jnp.zeros((16,), jnp.float32)
        return 0
    lax.fori_loop(0, STRIPE2 // 16, _zero_sb, 0)

    pltpu.sync_copy(nodeb, acc_sh.at[pl.ds(sid * STRIPE2, STRIPE2)])
    pltpu.sync_copy(sb, s_sh.at[pl.ds(sid * STRIPE2, STRIPE2)])
    plsc.subcore_barrier()

    for b in range(4):
        pltpu.async_copy(dst_hbm.at[sid * R1T + b], dstc.at[b], dsems[b])
        pltpu.async_copy(feat_hbm.at[srcb.at[b]], rings[b], sems[b])

    def pipe(i4, _):
        for b in range(4):
            r = i4 * 4 + b
            buf = rings[b]
            exb = exbufs[b]
            pltpu.make_async_copy(dst_hbm.at[sid * R1T + r], dstc.at[b],
                                  dsems[b]).wait()
            for g in range(8):
                s16 = srcb[r, pl.ds(g * 16, 16)]
                d16 = dstc[b, pl.ds(g * 16, 16)]
                ea = plsc.load_gather(el_v, [s16])
                eb = plsc.load_gather(er_v, [d16])
                e = ea + eb
                e = jnp.where(e >= 0.0, e, 0.2 * e)
                ex = jnp.exp(e)
                eid = (sid * R1T + r) * 128 + g * 16 + lax.iota(jnp.int32, 16)
                dl = d16 - lo
                mine = (dl >= 0) & (dl < NH2)
                ex = jnp.where((eid < E) & mine, ex, 0.0)
                exb[pl.ds(g * 16, 16)] = ex
                dstc[b, pl.ds(g * 16, 16)] = jnp.where(mine, dl, 0)
            pltpu.make_async_copy(feat_hbm.at[srcb.at[r]], buf, sems[b]).wait()

            def scale(g, _):
                exv = exb[pl.ds(g * 16, 16)]
                for t in range(16):
                    es = exv[t]
                    j = g * 16 + t
                    for q in range(CP // 16):
                        buf[j, pl.ds(q * 16, 16)] = buf[j, pl.ds(q * 16, 16)] * es
                return 0
            lax.fori_loop(0, 8, scale, 0)
            pltpu.sync_copy(buf, acc_sh.at[dstc.at[b]], add=True)
            pltpu.sync_copy(exb, s_sh.at[dstc.at[b]], add=True)

            @pl.when(r + 4 < R1T)
            def _():
                pltpu.async_copy(dst_hbm.at[sid * R1T + r + 4],
                                 dstc.at[b], dsems[b])
                pltpu.async_copy(feat_hbm.at[srcb.at[r + 4]], buf, sems[b])
        return 0
    lax.fori_loop(0, R1T // 4, pipe, 0)
    plsc.subcore_barrier()

    r0 = sid * STRIPE2
    pltpu.sync_copy(acc_sh.at[pl.ds(r0, STRIPE2)], nodeb)
    pltpu.sync_copy(s_sh.at[pl.ds(r0, STRIPE2)], sb)

    def fin(g, _):
        invv = 1.0 / (sb[pl.ds(g * 16, 16)] + 1e-9)
        for t in range(16):
            inv = invv[t]
            j = g * 16 + t
            for q in range(CP // 16):
                nodeb[j, pl.ds(q * 16, 16)] = (
                    nodeb[j, pl.ds(q * 16, 16)] * inv + b_v[pl.ds(q * 16, 16)])
        return 0
    lax.fori_loop(0, STRIPE2 // 16, fin, 0)
    pltpu.sync_copy(nodeb, out_hbm.at[pl.ds(lo + r0, STRIPE2)])


def _sc_layer2(feat2, el2, er2, b2p, srcR, dstR):
    fn = pl.kernel(
        _sc2_body,
        out_type=jax.ShapeDtypeStruct((NPAD, CP), jnp.float32),
        mesh=_mesh,
        compiler_params=pltpu.CompilerParams(needs_layout_passes=False, use_tc_tiling_on_sc=False),
        scratch_types=[
            pltpu.VMEM((R1T, 128), jnp.int32),
            pltpu.VMEM((4, 128), jnp.int32),
            pltpu.VMEM((NPAD,), jnp.float32),
            pltpu.VMEM((NPAD,), jnp.float32),
            pltpu.VMEM((CP,), jnp.float32),
            pltpu.VMEM((128,), jnp.float32),
            pltpu.VMEM((128,), jnp.float32),
            pltpu.VMEM((128,), jnp.float32),
            pltpu.VMEM((128,), jnp.float32),
            pltpu.VMEM((128, CP), jnp.float32),
            pltpu.VMEM((128, CP), jnp.float32),
            pltpu.VMEM((128, CP), jnp.float32),
            pltpu.VMEM((128, CP), jnp.float32),
            pltpu.VMEM((STRIPE2, CP), jnp.float32),
            pltpu.VMEM((STRIPE2,), jnp.float32),
            pltpu.VMEM_SHARED((NH2, CP), jnp.float32),
            pltpu.VMEM_SHARED((NH2,), jnp.float32),
            pltpu.SemaphoreType.DMA,
            pltpu.SemaphoreType.DMA,
            pltpu.SemaphoreType.DMA,
            pltpu.SemaphoreType.DMA,
            pltpu.SemaphoreType.DMA,
            pltpu.SemaphoreType.DMA,
            pltpu.SemaphoreType.DMA,
            pltpu.SemaphoreType.DMA,
        ],
    )
    return fn(feat2, el2, er2, b2p, srcR, dstR)


def kernel(features, edge_index, W1, al1, ar1, b1, W2, al2, ar2, b2):
    xp = jnp.pad(features, ((0, NPAD - N), (0, 0)))
    featH, elH, erH = _dense1(xp, W1, al1, ar1)

    src = edge_index[0]
    dst = edge_index[1]
    srcR = jnp.pad(src, (0, EPAD - E)).reshape(ROWS, 128)
    dstR = jnp.pad(dst, (0, EPAD - E)).reshape(ROWS, 128)

    out1F = _sc_layer1(featH.reshape(H * NPAD, D), elH.reshape(-1),
                       erH.reshape(-1), b1, srcR, dstR)

    W2p = jnp.pad(W2, ((0, 0), (0, CP - C)))
    al2p = jnp.pad(al2.reshape(1, C), ((0, 0), (0, CP - C)))
    ar2p = jnp.pad(ar2.reshape(1, C), ((0, 0), (0, CP - C)))
    feat2, elr2 = _dense2(out1F, W2p, al2p, ar2p)

    b2p = jnp.pad(b2.reshape(C), (0, CP - C))
    out = _sc_layer2(feat2, elr2[0], elr2[1], b2p, srcR, dstR)
    return out[:N, :C]
